# baseline (speedup 1.0000x reference)
import jax
import jax.numpy as jnp
from jax.experimental import pallas as pl
from jax.experimental.pallas import tpu as pltpu

VH, VW, VZ, D = 128, 128, 16, 128
N = VH * VW * VZ
BM = 64


def _body(prop_full_ref, prop_ref, L_ref, Wq_ref, bq_ref, W1T_ref, b1_ref,
          g_ref, beta_ref, W2_ref, b2_ref, out_ref, tot_ref):
    @pl.when(pl.program_id(0) == 0)
    def _():
        tot_ref[0, 0] = jnp.sum(prop_full_ref[...])

    L = L_ref[...].reshape(D, BM * VW)
    seed = jax.lax.dot_general(Wq_ref[...], L, (((0,), (0,)), ((), ())),
                               preferred_element_type=jnp.float32) + bq_ref[...][:, None]
    h = jax.lax.dot_general(W1T_ref[...], L, (((1,), (0,)), ((), ())),
                            preferred_element_type=jnp.float32) + b1_ref[...][:, None]
    m = jnp.mean(h, axis=0, keepdims=True)
    hc = h - m
    var = jnp.mean(hc * hc, axis=0, keepdims=True)
    hn = hc / jnp.sqrt(var + 1e-5) * g_ref[...][:, None] + beta_ref[...][:, None]
    hn = jnp.where(hn >= 0, hn, 0.01 * hn)
    prior = jax.lax.dot_general(W2_ref[...], hn, (((0,), (0,)), ((), ())),
                                preferred_element_type=jnp.float32) + b2_ref[...][:, None]
    unmasked = jnp.logical_or(prop_ref[0].astype(jnp.int32) > 0,
                              tot_ref[0, 0] < 2)
    out_ref[...] = jnp.where(unmasked, seed, prior).reshape(D, BM, VW)


def kernel(mlvl_feats, proposal, cam_params, lss_volume, W_q, b_q,
           W1, b1, ln_g, ln_b, W2, b2):
    L = lss_volume.transpose(0, 1, 2, 4, 3).reshape(D, VH * VZ, VW)
    prop_perm = (proposal.astype(jnp.int8).reshape(VH, VW, VZ)
                 .transpose(0, 2, 1).reshape(VH * VZ // BM, 1, BM * VW))
    out = pl.pallas_call(
        _body,
        grid=(VH * VZ // BM,),
        in_specs=[
            pl.BlockSpec((N // 128, 128), lambda i: (0, 0)),
            pl.BlockSpec((1, 1, BM * VW), lambda i: (i, 0, 0)),
            pl.BlockSpec((D, BM, VW), lambda i: (0, i, 0)),
            pl.BlockSpec((D, D), lambda i: (0, 0)),
            pl.BlockSpec((D,), lambda i: (0,)),
            pl.BlockSpec((D // 2, D), lambda i: (0, 0)),
            pl.BlockSpec((D // 2,), lambda i: (0,)),
            pl.BlockSpec((D // 2,), lambda i: (0,)),
            pl.BlockSpec((D // 2,), lambda i: (0,)),
            pl.BlockSpec((D // 2, D), lambda i: (0, 0)),
            pl.BlockSpec((D,), lambda i: (0,)),
        ],
        out_specs=pl.BlockSpec((D, BM, VW), lambda i: (0, i, 0)),
        out_shape=jax.ShapeDtypeStruct((D, VH * VZ, VW), jnp.float32),
        scratch_shapes=[pltpu.SMEM((1, 1), jnp.int32)],
        compiler_params=pltpu.CompilerParams(
            dimension_semantics=("arbitrary",)),
    )(proposal.reshape(N // 128, 128), prop_perm, L, W_q, b_q, W1.T,
      b1, ln_g, ln_b, W2, b2)
    return out.reshape(1, D, VH, VZ, VW).transpose(0, 1, 2, 4, 3)


# speedup vs baseline: 1.0181x; 1.0181x over previous
import jax
import jax.numpy as jnp
from jax.experimental import pallas as pl
from jax.experimental.pallas import tpu as pltpu

VH, VW, VZ, D = 128, 128, 16, 128
N = VH * VW * VZ
BM = 128


def _body(prop_full_ref, prop_ref, L_ref, Wq_ref, bq_ref, W1T_ref, b1_ref,
          g_ref, beta_ref, W2_ref, b2_ref, out_ref, tot_ref):
    @pl.when(pl.program_id(0) == 0)
    def _():
        tot_ref[0, 0] = jnp.sum(prop_full_ref[...])

    L = L_ref[...].reshape(D, BM * VW)
    seed = jax.lax.dot_general(Wq_ref[...], L, (((0,), (0,)), ((), ())),
                               preferred_element_type=jnp.float32) + bq_ref[...][:, None]
    h = jax.lax.dot_general(W1T_ref[...], L, (((1,), (0,)), ((), ())),
                            preferred_element_type=jnp.float32) + b1_ref[...][:, None]
    m = jnp.mean(h, axis=0, keepdims=True)
    hc = h - m
    var = jnp.mean(hc * hc, axis=0, keepdims=True)
    hn = hc / jnp.sqrt(var + 1e-5) * g_ref[...][:, None] + beta_ref[...][:, None]
    hn = jnp.where(hn >= 0, hn, 0.01 * hn)
    prior = jax.lax.dot_general(W2_ref[...], hn, (((0,), (0,)), ((), ())),
                                preferred_element_type=jnp.float32) + b2_ref[...][:, None]
    unmasked = jnp.logical_or(prop_ref[0].astype(jnp.int32) > 0,
                              tot_ref[0, 0] < 2)
    out_ref[...] = jnp.where(unmasked, seed, prior).reshape(D, BM, VW)


def kernel(mlvl_feats, proposal, cam_params, lss_volume, W_q, b_q,
           W1, b1, ln_g, ln_b, W2, b2):
    L = lss_volume.transpose(0, 1, 2, 4, 3).reshape(D, VH * VZ, VW)
    prop_perm = (proposal.astype(jnp.int8).reshape(VH, VW, VZ)
                 .transpose(0, 2, 1).reshape(VH * VZ // BM, 1, BM * VW))
    out = pl.pallas_call(
        _body,
        grid=(VH * VZ // BM,),
        in_specs=[
            pl.BlockSpec((N // 128, 128), lambda i: (0, 0)),
            pl.BlockSpec((1, 1, BM * VW), lambda i: (i, 0, 0)),
            pl.BlockSpec((D, BM, VW), lambda i: (0, i, 0)),
            pl.BlockSpec((D, D), lambda i: (0, 0)),
            pl.BlockSpec((D,), lambda i: (0,)),
            pl.BlockSpec((D // 2, D), lambda i: (0, 0)),
            pl.BlockSpec((D // 2,), lambda i: (0,)),
            pl.BlockSpec((D // 2,), lambda i: (0,)),
            pl.BlockSpec((D // 2,), lambda i: (0,)),
            pl.BlockSpec((D // 2, D), lambda i: (0, 0)),
            pl.BlockSpec((D,), lambda i: (0,)),
        ],
        out_specs=pl.BlockSpec((D, BM, VW), lambda i: (0, i, 0)),
        out_shape=jax.ShapeDtypeStruct((D, VH * VZ, VW), jnp.float32),
        scratch_shapes=[pltpu.SMEM((1, 1), jnp.int32)],
        compiler_params=pltpu.CompilerParams(
            dimension_semantics=("arbitrary",)),
    )(proposal.reshape(N // 128, 128), prop_perm, L, W_q, b_q, W1.T,
      b1, ln_g, ln_b, W2, b2)
    return out.reshape(1, D, VH, VZ, VW).transpose(0, 1, 2, 4, 3)
